# baseline (speedup 1.0000x reference)
import functools

import jax
import jax.numpy as jnp
from jax import lax
from jax.experimental import pallas as pl
from jax.experimental.pallas import tpu as pltpu
from jax.experimental.pallas import tpu_sc as plsc

F32 = jnp.float32
I32 = jnp.int32
NNODE = 10000
NP = 10240
BN = 512
NBLK = NP // BN
NGRAPH = 64

NC = 2
NS = 16
NW = NC * NS
NB = 64
RB = 160
EDGES = 320000
ECH = 2000
SBUF = 4096
SPAD = 176
CAPK = EDGES + SBUF + SPAD + 16

_sc_mesh = plsc.VectorSubcoreMesh(
    core_axis_name="c", subcore_axis_name="s", num_cores=NC, num_subcores=NS)


def _wid():
    return lax.axis_index("s") * NC + lax.axis_index("c")


def _bucket_body(src_hbm, dst_hbm, w_hbm, keys_hbm, wts_hbm, cnt_hbm,
                 srcv, dstv, wv, kstg0, wstg0, kstg1, wstg1, cbuf):
    wid = _wid()
    los = (2 * wid * RB, (2 * wid + 1) * RB)
    stgs = ((kstg0, wstg0), (kstg1, wstg1))

    def vreg_step(i, carry):
        p0, h0, p1, h1 = carry
        d = dstv[pl.ds(i * 16, 16)]
        s = srcv[pl.ds(i * 16, 16)]
        w = wv[pl.ds(i * 16, 16)]
        out = []
        for bi, (p, h) in enumerate(((p0, h0), (p1, h1))):
            lo = los[bi]
            kstg, wstg = stgs[bi]
            m = (d >= lo) & (d < lo + RB)
            key = (s << 8) | (d - lo)
            plsc.store_compressed(kstg.at[pl.ds(p, 16)], key, mask=m)
            plsc.store_compressed(wstg.at[pl.ds(p, 16)], w, mask=m)
            p = p + jnp.sum(m.astype(I32))

            def flush(args, kstg=kstg, wstg=wstg, bi=bi):
                p, h = args
                hb = pl.multiple_of((2 * wid + bi) * CAPK + h, 8)
                pltpu.sync_copy(kstg.at[pl.ds(0, SBUF)],
                                keys_hbm.at[pl.ds(hb, SBUF)])
                pltpu.sync_copy(wstg.at[pl.ds(0, SBUF)],
                                wts_hbm.at[pl.ds(hb, SBUF)])
                tk = kstg[pl.ds(SBUF, 16)]
                tw = wstg[pl.ds(SBUF, 16)]
                kstg[pl.ds(0, 16)] = tk
                wstg[pl.ds(0, 16)] = tw
                return p - SBUF, h + SBUF

            p, h = lax.cond(p >= SBUF, flush, lambda a: a, (p, h))
            out.extend((p, h))
        return tuple(out)

    def chunk_step(ci, carry):
        base = ci * ECH
        pltpu.sync_copy(src_hbm.at[pl.ds(base, ECH)], srcv)
        pltpu.sync_copy(dst_hbm.at[pl.ds(base, ECH)], dstv)
        pltpu.sync_copy(w_hbm.at[pl.ds(base, ECH)], wv)
        return lax.fori_loop(0, ECH // 16, vreg_step, carry)

    z = jnp.zeros((), I32)
    p0, h0, p1, h1 = lax.fori_loop(0, EDGES // ECH, chunk_step,
                                   (z, z, z, z))
    for bi, (p, h) in enumerate(((p0, h0), (p1, h1))):
        b = 2 * wid + bi
        kstg, wstg = stgs[bi]
        for t in range(SPAD // 16):
            kstg[pl.ds(p + 16 * t, 16)] = jnp.zeros((16,), I32)
            wstg[pl.ds(p + 16 * t, 16)] = jnp.zeros((16,), F32)
        pltpu.sync_copy(kstg.at[:],
                        keys_hbm.at[pl.ds(pl.multiple_of(b * CAPK + h, 8), SBUF + SPAD + 16)])
        pltpu.sync_copy(wstg.at[:],
                        wts_hbm.at[pl.ds(pl.multiple_of(b * CAPK + h, 8), SBUF + SPAD + 16)])
        cbuf[...] = jnp.full((16,), h + p, I32)
        pltpu.sync_copy(cbuf, cnt_hbm.at[pl.ds(pl.multiple_of(b * 16, 8), 16)])


_bucket_kernel = functools.partial(
    pl.kernel,
    out_type=[
        jax.ShapeDtypeStruct((NB * CAPK,), I32),
        jax.ShapeDtypeStruct((NB * CAPK,), F32),
        jax.ShapeDtypeStruct((NB * 16,), I32),
    ],
    mesh=_sc_mesh,
    scratch_types=[
        pltpu.VMEM((ECH,), I32),
        pltpu.VMEM((ECH,), I32),
        pltpu.VMEM((ECH,), F32),
        pltpu.VMEM((SBUF + SPAD + 16,), I32),
        pltpu.VMEM((SBUF + SPAD + 16,), F32),
        pltpu.VMEM((SBUF + SPAD + 16,), I32),
        pltpu.VMEM((SBUF + SPAD + 16,), F32),
        pltpu.VMEM((16,), I32),
    ],
    compiler_params=pltpu.CompilerParams(needs_layout_passes=False),
)(_bucket_body)


def _agg_body(Wp, C, NPASS, t_hbm, keys_hbm, wts_hbm, cnt_hbm, *rest):
    outs = rest[:NPASS]
    aggv, rowsv, keyv, wtv, idxv, cntv, sem = rest[NPASS:]
    wid = _wid()
    pltpu.sync_copy(cnt_hbm, cntv)

    def bucket_loop(bi, _):
        b = 2 * wid + bi
        cnt = cntv[pl.ds(pl.multiple_of(b * 16, 8), 16)][0]
        nch = lax.shift_right_logical(cnt + (C - 1), C.bit_length() - 1)
        for p in range(NPASS):

            def zb(r, _):
                for j in range(8):
                    aggv[pl.ds(128 * r + 16 * j, 16)] = jnp.zeros((16,), F32)
                return 0

            lax.fori_loop(0, RB * Wp // 128, zb, 0)

            def chunk_body(ci, _):
                base = pl.multiple_of(b * CAPK + ci * C, 8)
                pltpu.sync_copy(keys_hbm.at[pl.ds(base, C)], keyv)
                pltpu.sync_copy(wts_hbm.at[pl.ds(base, C)], wtv)
                for v in range(C // 16):
                    kv = keyv[pl.ds(16 * v, 16)]
                    idxv[pl.ds(16 * v, 16)] = (
                        lax.shift_right_logical(kv, 8) * NPASS + p)
                pltpu.async_copy(t_hbm.at[idxv], rowsv, sem).wait()
                lane = lax.broadcasted_iota(I32, (16,), 0)
                for e in range(C):
                    ev = jnp.full((16,), e, I32)
                    kv = plsc.load_gather(keyv, [ev])
                    wsp = plsc.load_gather(wtv, [ev])
                    basev = (kv & 255) * Wp + lane
                    for j in range(Wp // 16):
                        rv = rowsv[e, pl.ds(16 * j, 16)]
                        plsc.addupdate_scatter(aggv, [basev + 16 * j],
                                               rv * wsp)
                return 0

            lax.fori_loop(0, nch, chunk_body, 0)
            pltpu.sync_copy(
                aggv,
                outs[p].at[pl.ds(pl.multiple_of(b * RB * Wp, 8), RB * Wp)])
        return 0

    lax.fori_loop(0, 2, bucket_loop, 0)


@functools.lru_cache(maxsize=None)
def _make_agg(W):
    NPASS = 2 if W == 512 else 1
    Wp = W // NPASS
    C = (32 if NPASS > 1 else 64) if Wp == 256 else 128
    out_t = [jax.ShapeDtypeStruct((NP * Wp,), F32) for _ in range(NPASS)]
    return functools.partial(
        pl.kernel,
        out_type=out_t if NPASS > 1 else out_t[0],
        mesh=_sc_mesh,
        scratch_types=[
            pltpu.VMEM((RB * Wp,), F32),
            pltpu.VMEM((C, Wp), F32),
            pltpu.VMEM((C,), I32),
            pltpu.VMEM((C,), F32),
            pltpu.VMEM((C,), I32),
            pltpu.VMEM((NB * 16,), I32),
            pltpu.SemaphoreType.DMA,
        ],
        compiler_params=pltpu.CompilerParams(needs_layout_passes=False),
    )(functools.partial(_agg_body, Wp, C, NPASS))


def _layer2_body(a0_ref, a1_ref, h_ref, wr_ref, ws_ref, br_ref, out_ref):
    acc = jnp.dot(a0_ref[...], wr_ref[0:256, :], preferred_element_type=F32)
    acc += jnp.dot(a1_ref[...], wr_ref[256:512, :], preferred_element_type=F32)
    acc += jnp.dot(h_ref[...], ws_ref[...], preferred_element_type=F32)
    out_ref[...] = jnp.maximum(acc + br_ref[...], 0.0)


def _layer_full2(a0, a1, h, Wr, Ws, br2d):
    cin, cout = Wr.shape
    return pl.pallas_call(
        _layer2_body,
        grid=(NBLK,),
        in_specs=[
            pl.BlockSpec((BN, 256), lambda i: (i, 0)),
            pl.BlockSpec((BN, 256), lambda i: (i, 0)),
            pl.BlockSpec((BN, cin), lambda i: (i, 0)),
            pl.BlockSpec((cin, cout), lambda i: (0, 0)),
            pl.BlockSpec((cin, cout), lambda i: (0, 0)),
            pl.BlockSpec((1, cout), lambda i: (0, 0)),
        ],
        out_specs=pl.BlockSpec((BN, cout), lambda i: (i, 0)),
        out_shape=jax.ShapeDtypeStruct((NP, cout), F32),
    )(a0, a1, h, Wr, Ws, br2d)


def _layer_body(agg_ref, h_ref, wr_ref, ws_ref, br_ref, out_ref):
    acc = jnp.dot(agg_ref[...], wr_ref[...], preferred_element_type=F32)
    acc += jnp.dot(h_ref[...], ws_ref[...], preferred_element_type=F32)
    out_ref[...] = jnp.maximum(acc + br_ref[...], 0.0)


def _layer_full(agg, h, Wr, Ws, br2d):
    cin, cout = Wr.shape
    return pl.pallas_call(
        _layer_body,
        grid=(NBLK,),
        in_specs=[
            pl.BlockSpec((BN, cin), lambda i: (i, 0)),
            pl.BlockSpec((BN, cin), lambda i: (i, 0)),
            pl.BlockSpec((cin, cout), lambda i: (0, 0)),
            pl.BlockSpec((cin, cout), lambda i: (0, 0)),
            pl.BlockSpec((1, cout), lambda i: (0, 0)),
        ],
        out_specs=pl.BlockSpec((BN, cout), lambda i: (i, 0)),
        out_shape=jax.ShapeDtypeStruct((NP, cout), F32),
    )(agg, h, Wr, Ws, br2d)


def _pre_body(a_ref, b_ref, wr_ref, ws_ref, br_ref, t_ref, r_ref):
    h = jnp.maximum(a_ref[...] + b_ref[...], 0.0)
    t_ref[...] = jnp.dot(h, wr_ref[...], preferred_element_type=F32)
    r_ref[...] = jnp.dot(h, ws_ref[...], preferred_element_type=F32) + br_ref[...]


def _pre(a, b, Wr, Ws, br2d):
    cin, cout = Wr.shape
    return pl.pallas_call(
        _pre_body,
        grid=(NBLK,),
        in_specs=[
            pl.BlockSpec((BN, cin), lambda i: (i, 0)),
            pl.BlockSpec((BN, cin), lambda i: (i, 0)),
            pl.BlockSpec((cin, cout), lambda i: (0, 0)),
            pl.BlockSpec((cin, cout), lambda i: (0, 0)),
            pl.BlockSpec((1, cout), lambda i: (0, 0)),
        ],
        out_specs=[
            pl.BlockSpec((BN, cout), lambda i: (i, 0)),
            pl.BlockSpec((BN, cout), lambda i: (i, 0)),
        ],
        out_shape=[
            jax.ShapeDtypeStruct((NP, cout), F32),
            jax.ShapeDtypeStruct((NP, cout), F32),
        ],
    )(a, b, Wr, Ws, br2d)


def _pool_body(a_ref, b_ref, batch_ref, sums_ref, cnts_ref):
    i = pl.program_id(0)

    @pl.when(i == 0)
    def _():
        sums_ref[...] = jnp.zeros_like(sums_ref)
        cnts_ref[...] = jnp.zeros_like(cnts_ref)

    h = jnp.maximum(a_ref[...] + b_ref[...], 0.0)
    bids = batch_ref[...]
    valid = (bids < NGRAPH).astype(F32)
    onehot = jnp.where(
        lax.broadcasted_iota(jnp.int32, (NGRAPH, BN), 0) == bids, 1.0, 0.0
    ).astype(F32)
    h = jnp.where(bids.reshape(BN, 1) < NGRAPH, h, 0.0)
    sums_ref[...] += jnp.dot(onehot, h, preferred_element_type=F32)
    cnts_ref[...] += jnp.dot(
        onehot, jnp.broadcast_to(valid.reshape(BN, 1), (BN, 128)),
        preferred_element_type=F32)


def _pool(a, b, batch2d):
    return pl.pallas_call(
        _pool_body,
        grid=(NBLK,),
        in_specs=[
            pl.BlockSpec((BN, 128), lambda i: (i, 0)),
            pl.BlockSpec((BN, 128), lambda i: (i, 0)),
            pl.BlockSpec((1, BN), lambda i: (0, i)),
        ],
        out_specs=[
            pl.BlockSpec((NGRAPH, 128), lambda i: (0, 0)),
            pl.BlockSpec((NGRAPH, 128), lambda i: (0, 0)),
        ],
        out_shape=[
            jax.ShapeDtypeStruct((NGRAPH, 128), F32),
            jax.ShapeDtypeStruct((NGRAPH, 128), F32),
        ],
    )(a, b, batch2d)


def _head_body(sums_ref, cnts_ref, w0_ref, b0_ref, w1_ref, b1_ref, w2_ref,
               b2_ref, out_ref):
    g = sums_ref[...] / jnp.maximum(cnts_ref[...], 1.0)
    g = jnp.maximum(jnp.dot(g, w0_ref[...], preferred_element_type=F32)
                    + b0_ref[...], 0.0)
    g = jnp.maximum(jnp.dot(g, w1_ref[...], preferred_element_type=F32)
                    + b1_ref[...], 0.0)
    out_ref[...] = jnp.dot(g, w2_ref[...], preferred_element_type=F32) + b2_ref[...]


def _head(sums, cnts, MW0, Mb0, MW1p, Mb1p, MW2p, Mb2p):
    return pl.pallas_call(
        _head_body,
        out_shape=jax.ShapeDtypeStruct((NGRAPH, 128), F32),
    )(sums, cnts, MW0, Mb0, MW1p, Mb1p, MW2p, Mb2p)


def kernel(x, edge_index, edge_attr, batch, Wr0, br0, Ws0, Wr1, br1, Ws1,
           Wr2, br2, Ws2, Wr3, br3, Ws3, Wr4, br4, Ws4, MW0, Mb0, MW1, Mb1,
           MW2, Mb2):
    src, dst = edge_index[0], edge_index[1]
    x_pad = jnp.pad(x, ((0, NP - NNODE), (0, 0)))
    batch_pad = jnp.pad(batch, (0, NP - NNODE), constant_values=NGRAPH)
    batch2d = batch_pad.reshape(1, NP)

    keys, wts, cnts = _bucket_kernel(src, dst, edge_attr)

    def _aggregate(t, width):
        npass = 2 if width == 512 else 1
        tin = t.reshape(NP * npass, width // npass)
        out = _make_agg(width)(tin, keys, wts, cnts)
        return out

    h = x_pad
    for Wr, br, Ws in ((Wr0, br0, Ws0), (Wr1, br1, Ws1)):
        agg = _aggregate(h, Wr.shape[0]).reshape(NP, Wr.shape[0])
        h = _layer_full(agg, h, Wr, Ws, br.reshape(1, -1))
    agg2 = _aggregate(h, 512)
    h = _layer_full2(agg2[0].reshape(NP, 256), agg2[1].reshape(NP, 256), h, Wr2, Ws2, br2.reshape(1, -1))

    zeros3 = jnp.zeros_like(h)
    t3, r3 = _pre(h, zeros3, Wr3, Ws3, br3.reshape(1, -1))
    aggw3 = _aggregate(t3, Wr3.shape[1]).reshape(NP, Wr3.shape[1])
    t4, r4 = _pre(aggw3, r3, Wr4, Ws4, br4.reshape(1, -1))
    aggw4 = _aggregate(t4, Wr4.shape[1]).reshape(NP, Wr4.shape[1])

    sums, cnts = _pool(aggw4, r4, batch2d)
    MW1p = jnp.pad(MW1, ((0, 0), (0, 64)))
    Mb1p = jnp.pad(Mb1, (0, 64)).reshape(1, 128)
    MW2p = jnp.pad(MW2, ((0, 64), (0, 127)))
    Mb2p = jnp.pad(Mb2, (0, 127)).reshape(1, 128)
    out = _head(sums, cnts, MW0, Mb0.reshape(1, 128), MW1p, Mb1p, MW2p, Mb2p)
    return out[:, :1]


# speedup vs baseline: 1.9267x; 1.9267x over previous
import functools

import jax
import jax.numpy as jnp
from jax import lax
from jax.experimental import pallas as pl
from jax.experimental.pallas import tpu as pltpu
from jax.experimental.pallas import tpu_sc as plsc

F32 = jnp.float32
I32 = jnp.int32
NNODE = 10000
NP = 10240
BN = 512
NBLK = NP // BN
NGRAPH = 64

NC = 2
NS = 16
NW = NC * NS
NB = 64
RB = 160
EDGES = 320000
ECH = 2000
SBUF = 4096
SPAD = 176
CAPK = EDGES + SBUF + SPAD + 16

_sc_mesh = plsc.VectorSubcoreMesh(
    core_axis_name="c", subcore_axis_name="s", num_cores=NC, num_subcores=NS)


def _wid():
    return lax.axis_index("s") * NC + lax.axis_index("c")


def _bucket_body(src_hbm, dst_hbm, w_hbm, keys_hbm, wts_hbm, cnt_hbm,
                 srcv, dstv, wv, kstg0, wstg0, kstg1, wstg1, cbuf):
    wid = _wid()
    los = (2 * wid * RB, (2 * wid + 1) * RB)
    stgs = ((kstg0, wstg0), (kstg1, wstg1))

    def vreg_step(i, carry):
        p0, h0, p1, h1 = carry
        d = dstv[pl.ds(i * 16, 16)]
        s = srcv[pl.ds(i * 16, 16)]
        w = wv[pl.ds(i * 16, 16)]
        out = []
        for bi, (p, h) in enumerate(((p0, h0), (p1, h1))):
            lo = los[bi]
            kstg, wstg = stgs[bi]
            m = (d >= lo) & (d < lo + RB)
            key = (s << 8) | (d - lo)
            plsc.store_compressed(kstg.at[pl.ds(p, 16)], key, mask=m)
            plsc.store_compressed(wstg.at[pl.ds(p, 16)], w, mask=m)
            p = p + jnp.sum(m.astype(I32))

            def flush(args, kstg=kstg, wstg=wstg, bi=bi):
                p, h = args
                hb = pl.multiple_of((2 * wid + bi) * CAPK + h, 8)
                pltpu.sync_copy(kstg.at[pl.ds(0, SBUF)],
                                keys_hbm.at[pl.ds(hb, SBUF)])
                pltpu.sync_copy(wstg.at[pl.ds(0, SBUF)],
                                wts_hbm.at[pl.ds(hb, SBUF)])
                tk = kstg[pl.ds(SBUF, 16)]
                tw = wstg[pl.ds(SBUF, 16)]
                kstg[pl.ds(0, 16)] = tk
                wstg[pl.ds(0, 16)] = tw
                return p - SBUF, h + SBUF

            p, h = lax.cond(p >= SBUF, flush, lambda a: a, (p, h))
            out.extend((p, h))
        return tuple(out)

    def chunk_step(ci, carry):
        base = ci * ECH
        pltpu.sync_copy(src_hbm.at[pl.ds(base, ECH)], srcv)
        pltpu.sync_copy(dst_hbm.at[pl.ds(base, ECH)], dstv)
        pltpu.sync_copy(w_hbm.at[pl.ds(base, ECH)], wv)
        return lax.fori_loop(0, ECH // 16, vreg_step, carry)

    z = jnp.zeros((), I32)
    p0, h0, p1, h1 = lax.fori_loop(0, EDGES // ECH, chunk_step,
                                   (z, z, z, z))
    for bi, (p, h) in enumerate(((p0, h0), (p1, h1))):
        b = 2 * wid + bi
        kstg, wstg = stgs[bi]
        for t in range(SPAD // 16):
            kstg[pl.ds(p + 16 * t, 16)] = jnp.zeros((16,), I32)
            wstg[pl.ds(p + 16 * t, 16)] = jnp.zeros((16,), F32)
        pltpu.sync_copy(kstg.at[:],
                        keys_hbm.at[pl.ds(pl.multiple_of(b * CAPK + h, 8), SBUF + SPAD + 16)])
        pltpu.sync_copy(wstg.at[:],
                        wts_hbm.at[pl.ds(pl.multiple_of(b * CAPK + h, 8), SBUF + SPAD + 16)])
        cbuf[...] = jnp.full((16,), h + p, I32)
        pltpu.sync_copy(cbuf, cnt_hbm.at[pl.ds(pl.multiple_of(b * 16, 8), 16)])


_bucket_kernel = functools.partial(
    pl.kernel,
    out_type=[
        jax.ShapeDtypeStruct((NB * CAPK,), I32),
        jax.ShapeDtypeStruct((NB * CAPK,), F32),
        jax.ShapeDtypeStruct((NB * 16,), I32),
    ],
    mesh=_sc_mesh,
    scratch_types=[
        pltpu.VMEM((ECH,), I32),
        pltpu.VMEM((ECH,), I32),
        pltpu.VMEM((ECH,), F32),
        pltpu.VMEM((SBUF + SPAD + 16,), I32),
        pltpu.VMEM((SBUF + SPAD + 16,), F32),
        pltpu.VMEM((SBUF + SPAD + 16,), I32),
        pltpu.VMEM((SBUF + SPAD + 16,), F32),
        pltpu.VMEM((16,), I32),
    ],
    compiler_params=pltpu.CompilerParams(needs_layout_passes=False),
)(_bucket_body)


def _agg_body(Wp, C, NPASS, t_hbm, keys_hbm, wts_hbm, cnt_hbm, *rest):
    outs = rest[:NPASS]
    aggv, rowsv, keyv, wtv, idxv, cntv, sem = rest[NPASS:]
    wid = _wid()
    pltpu.sync_copy(cnt_hbm, cntv)

    def bucket_loop(bi, _):
        b = 2 * wid + bi
        cnt = cntv[pl.ds(pl.multiple_of(b * 16, 8), 16)][0]
        nch = lax.shift_right_logical(cnt + (C - 1), C.bit_length() - 1)
        for p in range(NPASS):

            def zb(r, _):
                for j in range(8):
                    aggv[pl.ds(128 * r + 16 * j, 16)] = jnp.zeros((16,), F32)
                return 0

            lax.fori_loop(0, RB * Wp // 128, zb, 0)

            def chunk_body(ci, carry):
                lane = lax.broadcasted_iota(I32, (16,), 0)
                prev = carry[0]
                accs = list(carry[1:])
                J = Wp // 16
                base = pl.multiple_of(b * CAPK + ci * C, 8)
                pltpu.sync_copy(keys_hbm.at[pl.ds(base, C)], keyv)
                pltpu.sync_copy(wts_hbm.at[pl.ds(base, C)], wtv)
                for v in range(C // 16):
                    kv = keyv[pl.ds(16 * v, 16)]
                    idxv[pl.ds(16 * v, 16)] = (
                        lax.shift_right_logical(kv, 8) * NPASS + p)
                pltpu.async_copy(t_hbm.at[idxv], rowsv, sem).wait()
                for v in range(C // 16):
                    kv = keyv[pl.ds(16 * v, 16)]
                    wvv = wtv[pl.ds(16 * v, 16)]
                    dlv = kv & 255
                    for l in range(16):
                        e = 16 * v + l
                        dl_s = dlv[l]
                        w_s = wvv[l]

                        def fl(a, prev=prev):
                            pv = jnp.full((16,), prev, I32) * Wp + lane
                            for j in range(J):
                                plsc.addupdate_scatter(aggv, [pv + 16 * j],
                                                       a[j])
                            return tuple(
                                jnp.zeros((16,), F32) for _ in range(J))

                        accs = list(lax.cond(dl_s != prev, fl,
                                             lambda a: a, tuple(accs)))
                        for j in range(J):
                            accs[j] = accs[j] + rowsv[e, pl.ds(16 * j, 16)] * w_s
                        prev = dl_s
                return (prev, *accs)

            J = Wp // 16
            z16 = jnp.zeros((16,), F32)
            fin = lax.fori_loop(0, nch, chunk_body,
                                (jnp.zeros((), I32),) + (z16,) * J)
            lanef = lax.broadcasted_iota(I32, (16,), 0)
            pvf = jnp.full((16,), fin[0], I32) * Wp + lanef
            for j in range(J):
                plsc.addupdate_scatter(aggv, [pvf + 16 * j], fin[1 + j])
            pltpu.sync_copy(
                aggv,
                outs[p].at[pl.ds(pl.multiple_of(b * RB * Wp, 8), RB * Wp)])
        return 0

    lax.fori_loop(0, 2, bucket_loop, 0)


@functools.lru_cache(maxsize=None)
def _make_agg(W):
    NPASS = 2 if W == 512 else 1
    Wp = W // NPASS
    C = 32 if Wp == 256 else 64
    out_t = [jax.ShapeDtypeStruct((NP * Wp,), F32) for _ in range(NPASS)]
    return functools.partial(
        pl.kernel,
        out_type=out_t if NPASS > 1 else out_t[0],
        mesh=_sc_mesh,
        scratch_types=[
            pltpu.VMEM((RB * Wp,), F32),
            pltpu.VMEM((C, Wp), F32),
            pltpu.VMEM((C,), I32),
            pltpu.VMEM((C,), F32),
            pltpu.VMEM((C,), I32),
            pltpu.VMEM((NB * 16,), I32),
            pltpu.SemaphoreType.DMA,
        ],
        compiler_params=pltpu.CompilerParams(needs_layout_passes=False),
    )(functools.partial(_agg_body, Wp, C, NPASS))


def _layer2_body(a0_ref, a1_ref, h_ref, wr_ref, ws_ref, br_ref, out_ref):
    acc = jnp.dot(a0_ref[...], wr_ref[0:256, :], preferred_element_type=F32)
    acc += jnp.dot(a1_ref[...], wr_ref[256:512, :], preferred_element_type=F32)
    acc += jnp.dot(h_ref[...], ws_ref[...], preferred_element_type=F32)
    out_ref[...] = jnp.maximum(acc + br_ref[...], 0.0)


def _layer_full2(a0, a1, h, Wr, Ws, br2d):
    cin, cout = Wr.shape
    return pl.pallas_call(
        _layer2_body,
        grid=(NBLK,),
        in_specs=[
            pl.BlockSpec((BN, 256), lambda i: (i, 0)),
            pl.BlockSpec((BN, 256), lambda i: (i, 0)),
            pl.BlockSpec((BN, cin), lambda i: (i, 0)),
            pl.BlockSpec((cin, cout), lambda i: (0, 0)),
            pl.BlockSpec((cin, cout), lambda i: (0, 0)),
            pl.BlockSpec((1, cout), lambda i: (0, 0)),
        ],
        out_specs=pl.BlockSpec((BN, cout), lambda i: (i, 0)),
        out_shape=jax.ShapeDtypeStruct((NP, cout), F32),
    )(a0, a1, h, Wr, Ws, br2d)


def _layer_body(agg_ref, h_ref, wr_ref, ws_ref, br_ref, out_ref):
    acc = jnp.dot(agg_ref[...], wr_ref[...], preferred_element_type=F32)
    acc += jnp.dot(h_ref[...], ws_ref[...], preferred_element_type=F32)
    out_ref[...] = jnp.maximum(acc + br_ref[...], 0.0)


def _layer_full(agg, h, Wr, Ws, br2d):
    cin, cout = Wr.shape
    return pl.pallas_call(
        _layer_body,
        grid=(NBLK,),
        in_specs=[
            pl.BlockSpec((BN, cin), lambda i: (i, 0)),
            pl.BlockSpec((BN, cin), lambda i: (i, 0)),
            pl.BlockSpec((cin, cout), lambda i: (0, 0)),
            pl.BlockSpec((cin, cout), lambda i: (0, 0)),
            pl.BlockSpec((1, cout), lambda i: (0, 0)),
        ],
        out_specs=pl.BlockSpec((BN, cout), lambda i: (i, 0)),
        out_shape=jax.ShapeDtypeStruct((NP, cout), F32),
    )(agg, h, Wr, Ws, br2d)


def _pre_body(a_ref, b_ref, wr_ref, ws_ref, br_ref, t_ref, r_ref):
    h = jnp.maximum(a_ref[...] + b_ref[...], 0.0)
    t_ref[...] = jnp.dot(h, wr_ref[...], preferred_element_type=F32)
    r_ref[...] = jnp.dot(h, ws_ref[...], preferred_element_type=F32) + br_ref[...]


def _pre(a, b, Wr, Ws, br2d):
    cin, cout = Wr.shape
    return pl.pallas_call(
        _pre_body,
        grid=(NBLK,),
        in_specs=[
            pl.BlockSpec((BN, cin), lambda i: (i, 0)),
            pl.BlockSpec((BN, cin), lambda i: (i, 0)),
            pl.BlockSpec((cin, cout), lambda i: (0, 0)),
            pl.BlockSpec((cin, cout), lambda i: (0, 0)),
            pl.BlockSpec((1, cout), lambda i: (0, 0)),
        ],
        out_specs=[
            pl.BlockSpec((BN, cout), lambda i: (i, 0)),
            pl.BlockSpec((BN, cout), lambda i: (i, 0)),
        ],
        out_shape=[
            jax.ShapeDtypeStruct((NP, cout), F32),
            jax.ShapeDtypeStruct((NP, cout), F32),
        ],
    )(a, b, Wr, Ws, br2d)


def _pool_body(a_ref, b_ref, batch_ref, sums_ref, cnts_ref):
    i = pl.program_id(0)

    @pl.when(i == 0)
    def _():
        sums_ref[...] = jnp.zeros_like(sums_ref)
        cnts_ref[...] = jnp.zeros_like(cnts_ref)

    h = jnp.maximum(a_ref[...] + b_ref[...], 0.0)
    bids = batch_ref[...]
    valid = (bids < NGRAPH).astype(F32)
    onehot = jnp.where(
        lax.broadcasted_iota(jnp.int32, (NGRAPH, BN), 0) == bids, 1.0, 0.0
    ).astype(F32)
    h = jnp.where(bids.reshape(BN, 1) < NGRAPH, h, 0.0)
    sums_ref[...] += jnp.dot(onehot, h, preferred_element_type=F32)
    cnts_ref[...] += jnp.dot(
        onehot, jnp.broadcast_to(valid.reshape(BN, 1), (BN, 128)),
        preferred_element_type=F32)


def _pool(a, b, batch2d):
    return pl.pallas_call(
        _pool_body,
        grid=(NBLK,),
        in_specs=[
            pl.BlockSpec((BN, 128), lambda i: (i, 0)),
            pl.BlockSpec((BN, 128), lambda i: (i, 0)),
            pl.BlockSpec((1, BN), lambda i: (0, i)),
        ],
        out_specs=[
            pl.BlockSpec((NGRAPH, 128), lambda i: (0, 0)),
            pl.BlockSpec((NGRAPH, 128), lambda i: (0, 0)),
        ],
        out_shape=[
            jax.ShapeDtypeStruct((NGRAPH, 128), F32),
            jax.ShapeDtypeStruct((NGRAPH, 128), F32),
        ],
    )(a, b, batch2d)


def _head_body(sums_ref, cnts_ref, w0_ref, b0_ref, w1_ref, b1_ref, w2_ref,
               b2_ref, out_ref):
    g = sums_ref[...] / jnp.maximum(cnts_ref[...], 1.0)
    g = jnp.maximum(jnp.dot(g, w0_ref[...], preferred_element_type=F32)
                    + b0_ref[...], 0.0)
    g = jnp.maximum(jnp.dot(g, w1_ref[...], preferred_element_type=F32)
                    + b1_ref[...], 0.0)
    out_ref[...] = jnp.dot(g, w2_ref[...], preferred_element_type=F32) + b2_ref[...]


def _head(sums, cnts, MW0, Mb0, MW1p, Mb1p, MW2p, Mb2p):
    return pl.pallas_call(
        _head_body,
        out_shape=jax.ShapeDtypeStruct((NGRAPH, 128), F32),
    )(sums, cnts, MW0, Mb0, MW1p, Mb1p, MW2p, Mb2p)


def kernel(x, edge_index, edge_attr, batch, Wr0, br0, Ws0, Wr1, br1, Ws1,
           Wr2, br2, Ws2, Wr3, br3, Ws3, Wr4, br4, Ws4, MW0, Mb0, MW1, Mb1,
           MW2, Mb2):
    src, dst = edge_index[0], edge_index[1]
    x_pad = jnp.pad(x, ((0, NP - NNODE), (0, 0)))
    batch_pad = jnp.pad(batch, (0, NP - NNODE), constant_values=NGRAPH)
    batch2d = batch_pad.reshape(1, NP)

    keys, wts, cnts = _bucket_kernel(src, dst, edge_attr)

    def _aggregate(t, width):
        npass = 2 if width == 512 else 1
        tin = t.reshape(NP * npass, width // npass)
        out = _make_agg(width)(tin, keys, wts, cnts)
        return out

    h = x_pad
    for Wr, br, Ws in ((Wr0, br0, Ws0), (Wr1, br1, Ws1)):
        agg = _aggregate(h, Wr.shape[0]).reshape(NP, Wr.shape[0])
        h = _layer_full(agg, h, Wr, Ws, br.reshape(1, -1))
    agg2 = _aggregate(h, 512)
    h = _layer_full2(agg2[0].reshape(NP, 256), agg2[1].reshape(NP, 256), h, Wr2, Ws2, br2.reshape(1, -1))

    zeros3 = jnp.zeros_like(h)
    t3, r3 = _pre(h, zeros3, Wr3, Ws3, br3.reshape(1, -1))
    aggw3 = _aggregate(t3, Wr3.shape[1]).reshape(NP, Wr3.shape[1])
    t4, r4 = _pre(aggw3, r3, Wr4, Ws4, br4.reshape(1, -1))
    aggw4 = _aggregate(t4, Wr4.shape[1]).reshape(NP, Wr4.shape[1])

    sums, cnts = _pool(aggw4, r4, batch2d)
    MW1p = jnp.pad(MW1, ((0, 0), (0, 64)))
    Mb1p = jnp.pad(Mb1, (0, 64)).reshape(1, 128)
    MW2p = jnp.pad(MW2, ((0, 64), (0, 127)))
    Mb2p = jnp.pad(Mb2, (0, 127)).reshape(1, 128)
    out = _head(sums, cnts, MW0, Mb0.reshape(1, 128), MW1p, Mb1p, MW2p, Mb2p)
    return out[:, :1]


# speedup vs baseline: 2.1647x; 1.1235x over previous
import functools

import jax
import jax.numpy as jnp
from jax import lax
from jax.experimental import pallas as pl
from jax.experimental.pallas import tpu as pltpu
from jax.experimental.pallas import tpu_sc as plsc

F32 = jnp.float32
I32 = jnp.int32
NNODE = 10000
NP = 10240
BN = 512
NBLK = NP // BN
NGRAPH = 64

NC = 2
NS = 16
NW = NC * NS
NB = 64
RB = 160
EDGES = 320000
ECH = 2000
SBUF = 4096
SPAD = 528
CAPK = EDGES + SBUF + SPAD + 16

_sc_mesh = plsc.VectorSubcoreMesh(
    core_axis_name="c", subcore_axis_name="s", num_cores=NC, num_subcores=NS)


def _wid():
    return lax.axis_index("s") * NC + lax.axis_index("c")


def _bucket_body(src_hbm, dst_hbm, w_hbm, keys_hbm, wts_hbm, cnt_hbm,
                 srcv, dstv, wv, kstg0, wstg0, kstg1, wstg1, cbuf):
    wid = _wid()
    los = (2 * wid * RB, (2 * wid + 1) * RB)
    stgs = ((kstg0, wstg0), (kstg1, wstg1))

    def vreg_step(i, carry):
        p0, h0, p1, h1 = carry
        d = dstv[pl.ds(i * 16, 16)]
        s = srcv[pl.ds(i * 16, 16)]
        w = wv[pl.ds(i * 16, 16)]
        out = []
        for bi, (p, h) in enumerate(((p0, h0), (p1, h1))):
            lo = los[bi]
            kstg, wstg = stgs[bi]
            m = (d >= lo) & (d < lo + RB)
            key = (s << 8) | (d - lo)
            plsc.store_compressed(kstg.at[pl.ds(p, 16)], key, mask=m)
            plsc.store_compressed(wstg.at[pl.ds(p, 16)], w, mask=m)
            p = p + jnp.sum(m.astype(I32))

            def flush(args, kstg=kstg, wstg=wstg, bi=bi):
                p, h = args
                hb = pl.multiple_of((2 * wid + bi) * CAPK + h, 8)
                pltpu.sync_copy(kstg.at[pl.ds(0, SBUF)],
                                keys_hbm.at[pl.ds(hb, SBUF)])
                pltpu.sync_copy(wstg.at[pl.ds(0, SBUF)],
                                wts_hbm.at[pl.ds(hb, SBUF)])
                tk = kstg[pl.ds(SBUF, 16)]
                tw = wstg[pl.ds(SBUF, 16)]
                kstg[pl.ds(0, 16)] = tk
                wstg[pl.ds(0, 16)] = tw
                return p - SBUF, h + SBUF

            p, h = lax.cond(p >= SBUF, flush, lambda a: a, (p, h))
            out.extend((p, h))
        return tuple(out)

    def chunk_step(ci, carry):
        base = ci * ECH
        pltpu.sync_copy(src_hbm.at[pl.ds(base, ECH)], srcv)
        pltpu.sync_copy(dst_hbm.at[pl.ds(base, ECH)], dstv)
        pltpu.sync_copy(w_hbm.at[pl.ds(base, ECH)], wv)
        return lax.fori_loop(0, ECH // 16, vreg_step, carry)

    z = jnp.zeros((), I32)
    p0, h0, p1, h1 = lax.fori_loop(0, EDGES // ECH, chunk_step,
                                   (z, z, z, z))
    for bi, (p, h) in enumerate(((p0, h0), (p1, h1))):
        b = 2 * wid + bi
        kstg, wstg = stgs[bi]
        for t in range(SPAD // 16):
            kstg[pl.ds(p + 16 * t, 16)] = jnp.zeros((16,), I32)
            wstg[pl.ds(p + 16 * t, 16)] = jnp.zeros((16,), F32)
        pltpu.sync_copy(kstg.at[:],
                        keys_hbm.at[pl.ds(pl.multiple_of(b * CAPK + h, 8), SBUF + SPAD + 16)])
        pltpu.sync_copy(wstg.at[:],
                        wts_hbm.at[pl.ds(pl.multiple_of(b * CAPK + h, 8), SBUF + SPAD + 16)])
        cbuf[...] = jnp.full((16,), h + p, I32)
        pltpu.sync_copy(cbuf, cnt_hbm.at[pl.ds(pl.multiple_of(b * 16, 8), 16)])


_bucket_kernel = functools.partial(
    pl.kernel,
    out_type=[
        jax.ShapeDtypeStruct((NB * CAPK,), I32),
        jax.ShapeDtypeStruct((NB * CAPK,), F32),
        jax.ShapeDtypeStruct((NB * 16,), I32),
    ],
    mesh=_sc_mesh,
    scratch_types=[
        pltpu.VMEM((ECH,), I32),
        pltpu.VMEM((ECH,), I32),
        pltpu.VMEM((ECH,), F32),
        pltpu.VMEM((SBUF + SPAD + 16,), I32),
        pltpu.VMEM((SBUF + SPAD + 16,), F32),
        pltpu.VMEM((SBUF + SPAD + 16,), I32),
        pltpu.VMEM((SBUF + SPAD + 16,), F32),
        pltpu.VMEM((16,), I32),
    ],
    compiler_params=pltpu.CompilerParams(needs_layout_passes=False),
)(_bucket_body)


def _agg_body(Wp, C, NPASS, t_hbm, keys_hbm, wts_hbm, cnt_hbm, *rest):
    outs = rest[:NPASS]
    aggv, rowsv, keyv, wtv, idxv, cntv, sem = rest[NPASS:]
    wid = _wid()
    pltpu.sync_copy(cnt_hbm, cntv)

    def bucket_loop(bi, _):
        b = 2 * wid + bi
        cnt = cntv[pl.ds(pl.multiple_of(b * 16, 8), 16)][0]
        nch = lax.shift_right_logical(cnt + (C - 1), C.bit_length() - 1)
        for p in range(NPASS):

            def zb(r, _):
                for j in range(8):
                    aggv[pl.ds(128 * r + 16 * j, 16)] = jnp.zeros((16,), F32)
                return 0

            lax.fori_loop(0, RB * Wp // 128, zb, 0)

            def chunk_body(ci, carry):
                lane = lax.broadcasted_iota(I32, (16,), 0)
                J = Wp // 16
                base = pl.multiple_of(b * CAPK + ci * C, 8)
                ck = pltpu.async_copy(keys_hbm.at[pl.ds(base, C)], keyv, sem)
                cw = pltpu.async_copy(wts_hbm.at[pl.ds(base, C)], wtv, sem)
                ck.wait()
                cw.wait()

                def idx_grp(v, _):
                    kv = keyv[pl.ds(16 * v, 16)]
                    idxv[pl.ds(16 * v, 16)] = (
                        lax.shift_right_logical(kv, 8) * NPASS + p)
                    return 0

                lax.fori_loop(0, C // 16, idx_grp, 0)
                pltpu.async_copy(t_hbm.at[idxv], rowsv, sem).wait()

                def acc_grp(v, gcarry):
                    prev = gcarry[0]
                    accs = list(gcarry[1:])
                    kv = keyv[pl.ds(16 * v, 16)]
                    wvv = wtv[pl.ds(16 * v, 16)]
                    dlv = kv & 255
                    for l in range(16):
                        dl_s = dlv[l]
                        w_s = wvv[l]

                        def fl(a, prev=prev):
                            pv = jnp.full((16,), prev, I32) * Wp + lane
                            for j in range(J):
                                plsc.addupdate_scatter(aggv, [pv + 16 * j],
                                                       a[j])
                            return tuple(
                                jnp.zeros((16,), F32) for _ in range(J))

                        accs = list(lax.cond(dl_s != prev, fl,
                                             lambda a: a, tuple(accs)))
                        ebase = 16 * v + l
                        for j in range(J):
                            accs[j] = (accs[j]
                                       + rowsv[ebase, pl.ds(16 * j, 16)] * w_s)
                        prev = dl_s
                    return (prev, *accs)

                return lax.fori_loop(0, C // 16, acc_grp, carry)

            J = Wp // 16
            z16 = jnp.zeros((16,), F32)
            fin = lax.fori_loop(0, nch, chunk_body,
                                (jnp.zeros((), I32),) + (z16,) * J)
            lanef = lax.broadcasted_iota(I32, (16,), 0)
            pvf = jnp.full((16,), fin[0], I32) * Wp + lanef
            for j in range(J):
                plsc.addupdate_scatter(aggv, [pvf + 16 * j], fin[1 + j])
            pltpu.sync_copy(
                aggv,
                outs[p].at[pl.ds(pl.multiple_of(b * RB * Wp, 8), RB * Wp)])
        return 0

    lax.fori_loop(0, 2, bucket_loop, 0)


@functools.lru_cache(maxsize=None)
def _make_agg(W):
    NPASS = 2 if W == 512 else 1
    Wp = W // NPASS
    C = 256 if Wp == 256 else 512
    out_t = [jax.ShapeDtypeStruct((NP * Wp,), F32) for _ in range(NPASS)]
    return functools.partial(
        pl.kernel,
        out_type=out_t if NPASS > 1 else out_t[0],
        mesh=_sc_mesh,
        scratch_types=[
            pltpu.VMEM((RB * Wp,), F32),
            pltpu.VMEM((C, Wp), F32),
            pltpu.VMEM((C,), I32),
            pltpu.VMEM((C,), F32),
            pltpu.VMEM((C,), I32),
            pltpu.VMEM((NB * 16,), I32),
            pltpu.SemaphoreType.DMA,
        ],
        compiler_params=pltpu.CompilerParams(needs_layout_passes=False),
    )(functools.partial(_agg_body, Wp, C, NPASS))


def _layer2_body(a0_ref, a1_ref, h_ref, wr_ref, ws_ref, br_ref, out_ref):
    acc = jnp.dot(a0_ref[...], wr_ref[0:256, :], preferred_element_type=F32)
    acc += jnp.dot(a1_ref[...], wr_ref[256:512, :], preferred_element_type=F32)
    acc += jnp.dot(h_ref[...], ws_ref[...], preferred_element_type=F32)
    out_ref[...] = jnp.maximum(acc + br_ref[...], 0.0)


def _layer_full2(a0, a1, h, Wr, Ws, br2d):
    cin, cout = Wr.shape
    return pl.pallas_call(
        _layer2_body,
        grid=(NBLK,),
        in_specs=[
            pl.BlockSpec((BN, 256), lambda i: (i, 0)),
            pl.BlockSpec((BN, 256), lambda i: (i, 0)),
            pl.BlockSpec((BN, cin), lambda i: (i, 0)),
            pl.BlockSpec((cin, cout), lambda i: (0, 0)),
            pl.BlockSpec((cin, cout), lambda i: (0, 0)),
            pl.BlockSpec((1, cout), lambda i: (0, 0)),
        ],
        out_specs=pl.BlockSpec((BN, cout), lambda i: (i, 0)),
        out_shape=jax.ShapeDtypeStruct((NP, cout), F32),
    )(a0, a1, h, Wr, Ws, br2d)


def _layer_body(agg_ref, h_ref, wr_ref, ws_ref, br_ref, out_ref):
    acc = jnp.dot(agg_ref[...], wr_ref[...], preferred_element_type=F32)
    acc += jnp.dot(h_ref[...], ws_ref[...], preferred_element_type=F32)
    out_ref[...] = jnp.maximum(acc + br_ref[...], 0.0)


def _layer_full(agg, h, Wr, Ws, br2d):
    cin, cout = Wr.shape
    return pl.pallas_call(
        _layer_body,
        grid=(NBLK,),
        in_specs=[
            pl.BlockSpec((BN, cin), lambda i: (i, 0)),
            pl.BlockSpec((BN, cin), lambda i: (i, 0)),
            pl.BlockSpec((cin, cout), lambda i: (0, 0)),
            pl.BlockSpec((cin, cout), lambda i: (0, 0)),
            pl.BlockSpec((1, cout), lambda i: (0, 0)),
        ],
        out_specs=pl.BlockSpec((BN, cout), lambda i: (i, 0)),
        out_shape=jax.ShapeDtypeStruct((NP, cout), F32),
    )(agg, h, Wr, Ws, br2d)


def _pre_body(a_ref, b_ref, wr_ref, ws_ref, br_ref, t_ref, r_ref):
    h = jnp.maximum(a_ref[...] + b_ref[...], 0.0)
    t_ref[...] = jnp.dot(h, wr_ref[...], preferred_element_type=F32)
    r_ref[...] = jnp.dot(h, ws_ref[...], preferred_element_type=F32) + br_ref[...]


def _pre(a, b, Wr, Ws, br2d):
    cin, cout = Wr.shape
    return pl.pallas_call(
        _pre_body,
        grid=(NBLK,),
        in_specs=[
            pl.BlockSpec((BN, cin), lambda i: (i, 0)),
            pl.BlockSpec((BN, cin), lambda i: (i, 0)),
            pl.BlockSpec((cin, cout), lambda i: (0, 0)),
            pl.BlockSpec((cin, cout), lambda i: (0, 0)),
            pl.BlockSpec((1, cout), lambda i: (0, 0)),
        ],
        out_specs=[
            pl.BlockSpec((BN, cout), lambda i: (i, 0)),
            pl.BlockSpec((BN, cout), lambda i: (i, 0)),
        ],
        out_shape=[
            jax.ShapeDtypeStruct((NP, cout), F32),
            jax.ShapeDtypeStruct((NP, cout), F32),
        ],
    )(a, b, Wr, Ws, br2d)


def _pool_body(a_ref, b_ref, batch_ref, sums_ref, cnts_ref):
    i = pl.program_id(0)

    @pl.when(i == 0)
    def _():
        sums_ref[...] = jnp.zeros_like(sums_ref)
        cnts_ref[...] = jnp.zeros_like(cnts_ref)

    h = jnp.maximum(a_ref[...] + b_ref[...], 0.0)
    bids = batch_ref[...]
    valid = (bids < NGRAPH).astype(F32)
    onehot = jnp.where(
        lax.broadcasted_iota(jnp.int32, (NGRAPH, BN), 0) == bids, 1.0, 0.0
    ).astype(F32)
    h = jnp.where(bids.reshape(BN, 1) < NGRAPH, h, 0.0)
    sums_ref[...] += jnp.dot(onehot, h, preferred_element_type=F32)
    cnts_ref[...] += jnp.dot(
        onehot, jnp.broadcast_to(valid.reshape(BN, 1), (BN, 128)),
        preferred_element_type=F32)


def _pool(a, b, batch2d):
    return pl.pallas_call(
        _pool_body,
        grid=(NBLK,),
        in_specs=[
            pl.BlockSpec((BN, 128), lambda i: (i, 0)),
            pl.BlockSpec((BN, 128), lambda i: (i, 0)),
            pl.BlockSpec((1, BN), lambda i: (0, i)),
        ],
        out_specs=[
            pl.BlockSpec((NGRAPH, 128), lambda i: (0, 0)),
            pl.BlockSpec((NGRAPH, 128), lambda i: (0, 0)),
        ],
        out_shape=[
            jax.ShapeDtypeStruct((NGRAPH, 128), F32),
            jax.ShapeDtypeStruct((NGRAPH, 128), F32),
        ],
    )(a, b, batch2d)


def _head_body(sums_ref, cnts_ref, w0_ref, b0_ref, w1_ref, b1_ref, w2_ref,
               b2_ref, out_ref):
    g = sums_ref[...] / jnp.maximum(cnts_ref[...], 1.0)
    g = jnp.maximum(jnp.dot(g, w0_ref[...], preferred_element_type=F32)
                    + b0_ref[...], 0.0)
    g = jnp.maximum(jnp.dot(g, w1_ref[...], preferred_element_type=F32)
                    + b1_ref[...], 0.0)
    out_ref[...] = jnp.dot(g, w2_ref[...], preferred_element_type=F32) + b2_ref[...]


def _head(sums, cnts, MW0, Mb0, MW1p, Mb1p, MW2p, Mb2p):
    return pl.pallas_call(
        _head_body,
        out_shape=jax.ShapeDtypeStruct((NGRAPH, 128), F32),
    )(sums, cnts, MW0, Mb0, MW1p, Mb1p, MW2p, Mb2p)


def kernel(x, edge_index, edge_attr, batch, Wr0, br0, Ws0, Wr1, br1, Ws1,
           Wr2, br2, Ws2, Wr3, br3, Ws3, Wr4, br4, Ws4, MW0, Mb0, MW1, Mb1,
           MW2, Mb2):
    src, dst = edge_index[0], edge_index[1]
    x_pad = jnp.pad(x, ((0, NP - NNODE), (0, 0)))
    batch_pad = jnp.pad(batch, (0, NP - NNODE), constant_values=NGRAPH)
    batch2d = batch_pad.reshape(1, NP)

    keys, wts, cnts = _bucket_kernel(src, dst, edge_attr)

    def _aggregate(t, width):
        npass = 2 if width == 512 else 1
        tin = t.reshape(NP * npass, width // npass)
        out = _make_agg(width)(tin, keys, wts, cnts)
        return out

    h = x_pad
    for Wr, br, Ws in ((Wr0, br0, Ws0), (Wr1, br1, Ws1)):
        agg = _aggregate(h, Wr.shape[0]).reshape(NP, Wr.shape[0])
        h = _layer_full(agg, h, Wr, Ws, br.reshape(1, -1))
    agg2 = _aggregate(h, 512)
    h = _layer_full2(agg2[0].reshape(NP, 256), agg2[1].reshape(NP, 256), h, Wr2, Ws2, br2.reshape(1, -1))

    zeros3 = jnp.zeros_like(h)
    t3, r3 = _pre(h, zeros3, Wr3, Ws3, br3.reshape(1, -1))
    aggw3 = _aggregate(t3, Wr3.shape[1]).reshape(NP, Wr3.shape[1])
    t4, r4 = _pre(aggw3, r3, Wr4, Ws4, br4.reshape(1, -1))
    aggw4 = _aggregate(t4, Wr4.shape[1]).reshape(NP, Wr4.shape[1])

    sums, cnts = _pool(aggw4, r4, batch2d)
    MW1p = jnp.pad(MW1, ((0, 0), (0, 64)))
    Mb1p = jnp.pad(Mb1, (0, 64)).reshape(1, 128)
    MW2p = jnp.pad(MW2, ((0, 64), (0, 127)))
    Mb2p = jnp.pad(Mb2, (0, 127)).reshape(1, 128)
    out = _head(sums, cnts, MW0, Mb0.reshape(1, 128), MW1p, Mb1p, MW2p, Mb2p)
    return out[:, :1]


# speedup vs baseline: 2.9172x; 1.3476x over previous
import functools

import jax
import jax.numpy as jnp
from jax import lax
from jax.experimental import pallas as pl
from jax.experimental.pallas import tpu as pltpu
from jax.experimental.pallas import tpu_sc as plsc

F32 = jnp.float32
I32 = jnp.int32
NNODE = 10000
NP = 10240
BN = 512
NBLK = NP // BN
NGRAPH = 64

NC = 2
NS = 16
NW = NC * NS
NB = 64
RB = 160
EDGES = 320000
ECH = 2000
SBUF = 4096
SPAD = 528
CAPK = EDGES + SBUF + SPAD + 16

_sc_mesh = plsc.VectorSubcoreMesh(
    core_axis_name="c", subcore_axis_name="s", num_cores=NC, num_subcores=NS)


def _wid():
    return lax.axis_index("s") * NC + lax.axis_index("c")


def _bucket_body(src_hbm, dst_hbm, w_hbm, keys_hbm, wts_hbm, cnt_hbm,
                 srcv, dstv, wv, kstg0, wstg0, kstg1, wstg1, cbuf):
    wid = _wid()
    los = (2 * wid * RB, (2 * wid + 1) * RB)
    stgs = ((kstg0, wstg0), (kstg1, wstg1))

    def vreg_step(i, carry):
        p0, h0, p1, h1 = carry
        d = dstv[pl.ds(i * 16, 16)]
        s = srcv[pl.ds(i * 16, 16)]
        w = wv[pl.ds(i * 16, 16)]
        out = []
        for bi, (p, h) in enumerate(((p0, h0), (p1, h1))):
            lo = los[bi]
            kstg, wstg = stgs[bi]
            m = (d >= lo) & (d < lo + RB)
            key = (s << 8) | (d - lo)
            plsc.store_compressed(kstg.at[pl.ds(p, 16)], key, mask=m)
            plsc.store_compressed(wstg.at[pl.ds(p, 16)], w, mask=m)
            p = p + jnp.sum(m.astype(I32))

            def flush(args, kstg=kstg, wstg=wstg, bi=bi):
                p, h = args
                hb = pl.multiple_of((2 * wid + bi) * CAPK + h, 8)
                pltpu.sync_copy(kstg.at[pl.ds(0, SBUF)],
                                keys_hbm.at[pl.ds(hb, SBUF)])
                pltpu.sync_copy(wstg.at[pl.ds(0, SBUF)],
                                wts_hbm.at[pl.ds(hb, SBUF)])
                tk = kstg[pl.ds(SBUF, 16)]
                tw = wstg[pl.ds(SBUF, 16)]
                kstg[pl.ds(0, 16)] = tk
                wstg[pl.ds(0, 16)] = tw
                return p - SBUF, h + SBUF

            p, h = lax.cond(p >= SBUF, flush, lambda a: a, (p, h))
            out.extend((p, h))
        return tuple(out)

    def chunk_step(ci, carry):
        base = ci * ECH
        pltpu.sync_copy(src_hbm.at[pl.ds(base, ECH)], srcv)
        pltpu.sync_copy(dst_hbm.at[pl.ds(base, ECH)], dstv)
        pltpu.sync_copy(w_hbm.at[pl.ds(base, ECH)], wv)
        return lax.fori_loop(0, ECH // 16, vreg_step, carry)

    z = jnp.zeros((), I32)
    p0, h0, p1, h1 = lax.fori_loop(0, EDGES // ECH, chunk_step,
                                   (z, z, z, z))
    for bi, (p, h) in enumerate(((p0, h0), (p1, h1))):
        b = 2 * wid + bi
        kstg, wstg = stgs[bi]
        for t in range(SPAD // 16):
            kstg[pl.ds(p + 16 * t, 16)] = jnp.zeros((16,), I32)
            wstg[pl.ds(p + 16 * t, 16)] = jnp.zeros((16,), F32)
        pltpu.sync_copy(kstg.at[:],
                        keys_hbm.at[pl.ds(pl.multiple_of(b * CAPK + h, 8), SBUF + SPAD + 16)])
        pltpu.sync_copy(wstg.at[:],
                        wts_hbm.at[pl.ds(pl.multiple_of(b * CAPK + h, 8), SBUF + SPAD + 16)])
        cbuf[...] = jnp.full((16,), h + p, I32)
        pltpu.sync_copy(cbuf, cnt_hbm.at[pl.ds(pl.multiple_of(b * 16, 8), 16)])


_bucket_kernel = functools.partial(
    pl.kernel,
    out_type=[
        jax.ShapeDtypeStruct((NB * CAPK,), I32),
        jax.ShapeDtypeStruct((NB * CAPK,), F32),
        jax.ShapeDtypeStruct((NB * 16,), I32),
    ],
    mesh=_sc_mesh,
    scratch_types=[
        pltpu.VMEM((ECH,), I32),
        pltpu.VMEM((ECH,), I32),
        pltpu.VMEM((ECH,), F32),
        pltpu.VMEM((SBUF + SPAD + 16,), I32),
        pltpu.VMEM((SBUF + SPAD + 16,), F32),
        pltpu.VMEM((SBUF + SPAD + 16,), I32),
        pltpu.VMEM((SBUF + SPAD + 16,), F32),
        pltpu.VMEM((16,), I32),
    ],
    compiler_params=pltpu.CompilerParams(needs_layout_passes=False),
)(_bucket_body)


def _agg_body(Wp, C, NPASS, t_hbm, keys_hbm, wts_hbm, cnt_hbm, *rest):
    outs = rest[:NPASS]
    (aggv, rowsA, rowsB, keyA, wtA, idxA, keyB, wtB, idxB, cntv,
     semK, semA, semB) = rest[NPASS:]
    wid = _wid()
    J = Wp // 16
    pltpu.sync_copy(cnt_hbm, cntv)

    def bucket_loop(bi, _):
        b = 2 * wid + bi
        cnt = cntv[pl.ds(pl.multiple_of(b * 16, 8), 16)][0]
        nch = lax.shift_right_logical(cnt + (C - 1), C.bit_length() - 1)
        for p in range(NPASS):

            def zb(r, _):
                for j in range(8):
                    aggv[pl.ds(128 * r + 16 * j, 16)] = jnp.zeros((16,), F32)
                return 0

            lax.fori_loop(0, RB * Wp // 128, zb, 0)

            def stage(ci, kbuf, wbuf, ibuf, rbuf, gsem):
                base = pl.multiple_of(b * CAPK + ci * C, 8)
                ck = pltpu.async_copy(keys_hbm.at[pl.ds(base, C)], kbuf, semK)
                cw = pltpu.async_copy(wts_hbm.at[pl.ds(base, C)], wbuf, semK)
                ck.wait()
                cw.wait()

                def idx_grp(v, _):
                    kv = kbuf[pl.ds(16 * v, 16)]
                    ibuf[pl.ds(16 * v, 16)] = (
                        lax.shift_right_logical(kv, 8) * NPASS + p)
                    return 0

                lax.fori_loop(0, C // 16, idx_grp, 0)
                pltpu.async_copy(t_hbm.at[ibuf], rbuf, gsem)

            def make_acc(kbuf, wbuf, rbuf):
                def acc_grp(v, gcarry):
                    lane = lax.broadcasted_iota(I32, (16,), 0)
                    prev = gcarry[0]
                    accs = list(gcarry[1:])
                    kv = kbuf[pl.ds(16 * v, 16)]
                    wvv = wbuf[pl.ds(16 * v, 16)]
                    dlv = kv & 255
                    for l in range(16):
                        dl_s = dlv[l]
                        w_s = wvv[l]

                        def fl(a, prev=prev):
                            pv = jnp.full((16,), prev, I32) * Wp + lane
                            for j in range(J):
                                plsc.addupdate_scatter(aggv, [pv + 16 * j],
                                                       a[j])
                            return tuple(
                                jnp.zeros((16,), F32) for _ in range(J))

                        accs = list(lax.cond(dl_s != prev, fl,
                                             lambda a: a, tuple(accs)))
                        for j in range(J):
                            accs[j] = (accs[j]
                                       + rbuf[16 * v + l, pl.ds(16 * j, 16)]
                                       * w_s)
                        prev = dl_s
                    return (prev, *accs)

                return acc_grp

            @pl.when(nch > 0)
            def _():
                stage(0, keyA, wtA, idxA, rowsA, semA)

            def pair(g, carry):
                @pl.when(2 * g + 1 < nch)
                def _():
                    stage(2 * g + 1, keyB, wtB, idxB, rowsB, semB)

                pltpu.make_async_copy(t_hbm.at[idxA], rowsA, semA).wait()
                carry = lax.fori_loop(0, C // 16,
                                      make_acc(keyA, wtA, rowsA), carry)

                @pl.when(2 * g + 2 < nch)
                def _():
                    stage(2 * g + 2, keyA, wtA, idxA, rowsA, semA)

                def do_b(c):
                    pltpu.make_async_copy(t_hbm.at[idxB], rowsB, semB).wait()
                    return lax.fori_loop(0, C // 16,
                                         make_acc(keyB, wtB, rowsB), c)

                return lax.cond(2 * g + 1 < nch, do_b, lambda c: c, carry)

            z16 = jnp.zeros((16,), F32)
            fin = lax.fori_loop(0, lax.shift_right_logical(nch + 1, 1), pair,
                                (jnp.zeros((), I32),) + (z16,) * J)
            lanef = lax.broadcasted_iota(I32, (16,), 0)
            pvf = jnp.full((16,), fin[0], I32) * Wp + lanef
            for j in range(J):
                plsc.addupdate_scatter(aggv, [pvf + 16 * j], fin[1 + j])
            pltpu.sync_copy(
                aggv,
                outs[p].at[pl.ds(pl.multiple_of(b * RB * Wp, 8), RB * Wp)])
        return 0

    lax.fori_loop(0, 2, bucket_loop, 0)


@functools.lru_cache(maxsize=None)
def _make_agg(W):
    NPASS = 2 if W == 512 else 1
    Wp = W // NPASS
    C = 128 if Wp == 256 else 256
    out_t = [jax.ShapeDtypeStruct((NP * Wp,), F32) for _ in range(NPASS)]
    return functools.partial(
        pl.kernel,
        out_type=out_t if NPASS > 1 else out_t[0],
        mesh=_sc_mesh,
        scratch_types=[
            pltpu.VMEM((RB * Wp,), F32),
            pltpu.VMEM((C, Wp), F32),
            pltpu.VMEM((C, Wp), F32),
            pltpu.VMEM((C,), I32),
            pltpu.VMEM((C,), F32),
            pltpu.VMEM((C,), I32),
            pltpu.VMEM((C,), I32),
            pltpu.VMEM((C,), F32),
            pltpu.VMEM((C,), I32),
            pltpu.VMEM((NB * 16,), I32),
            pltpu.SemaphoreType.DMA,
            pltpu.SemaphoreType.DMA,
            pltpu.SemaphoreType.DMA,
        ],
        compiler_params=pltpu.CompilerParams(needs_layout_passes=False),
    )(functools.partial(_agg_body, Wp, C, NPASS))


def _layer2_body(a0_ref, a1_ref, h_ref, wr_ref, ws_ref, br_ref, out_ref):
    acc = jnp.dot(a0_ref[...], wr_ref[0:256, :], preferred_element_type=F32)
    acc += jnp.dot(a1_ref[...], wr_ref[256:512, :], preferred_element_type=F32)
    acc += jnp.dot(h_ref[...], ws_ref[...], preferred_element_type=F32)
    out_ref[...] = jnp.maximum(acc + br_ref[...], 0.0)


def _layer_full2(a0, a1, h, Wr, Ws, br2d):
    cin, cout = Wr.shape
    return pl.pallas_call(
        _layer2_body,
        grid=(NBLK,),
        in_specs=[
            pl.BlockSpec((BN, 256), lambda i: (i, 0)),
            pl.BlockSpec((BN, 256), lambda i: (i, 0)),
            pl.BlockSpec((BN, cin), lambda i: (i, 0)),
            pl.BlockSpec((cin, cout), lambda i: (0, 0)),
            pl.BlockSpec((cin, cout), lambda i: (0, 0)),
            pl.BlockSpec((1, cout), lambda i: (0, 0)),
        ],
        out_specs=pl.BlockSpec((BN, cout), lambda i: (i, 0)),
        out_shape=jax.ShapeDtypeStruct((NP, cout), F32),
    )(a0, a1, h, Wr, Ws, br2d)


def _layer_body(agg_ref, h_ref, wr_ref, ws_ref, br_ref, out_ref):
    acc = jnp.dot(agg_ref[...], wr_ref[...], preferred_element_type=F32)
    acc += jnp.dot(h_ref[...], ws_ref[...], preferred_element_type=F32)
    out_ref[...] = jnp.maximum(acc + br_ref[...], 0.0)


def _layer_full(agg, h, Wr, Ws, br2d):
    cin, cout = Wr.shape
    return pl.pallas_call(
        _layer_body,
        grid=(NBLK,),
        in_specs=[
            pl.BlockSpec((BN, cin), lambda i: (i, 0)),
            pl.BlockSpec((BN, cin), lambda i: (i, 0)),
            pl.BlockSpec((cin, cout), lambda i: (0, 0)),
            pl.BlockSpec((cin, cout), lambda i: (0, 0)),
            pl.BlockSpec((1, cout), lambda i: (0, 0)),
        ],
        out_specs=pl.BlockSpec((BN, cout), lambda i: (i, 0)),
        out_shape=jax.ShapeDtypeStruct((NP, cout), F32),
    )(agg, h, Wr, Ws, br2d)


def _pre_body(a_ref, b_ref, wr_ref, ws_ref, br_ref, t_ref, r_ref):
    h = jnp.maximum(a_ref[...] + b_ref[...], 0.0)
    t_ref[...] = jnp.dot(h, wr_ref[...], preferred_element_type=F32)
    r_ref[...] = jnp.dot(h, ws_ref[...], preferred_element_type=F32) + br_ref[...]


def _pre(a, b, Wr, Ws, br2d):
    cin, cout = Wr.shape
    return pl.pallas_call(
        _pre_body,
        grid=(NBLK,),
        in_specs=[
            pl.BlockSpec((BN, cin), lambda i: (i, 0)),
            pl.BlockSpec((BN, cin), lambda i: (i, 0)),
            pl.BlockSpec((cin, cout), lambda i: (0, 0)),
            pl.BlockSpec((cin, cout), lambda i: (0, 0)),
            pl.BlockSpec((1, cout), lambda i: (0, 0)),
        ],
        out_specs=[
            pl.BlockSpec((BN, cout), lambda i: (i, 0)),
            pl.BlockSpec((BN, cout), lambda i: (i, 0)),
        ],
        out_shape=[
            jax.ShapeDtypeStruct((NP, cout), F32),
            jax.ShapeDtypeStruct((NP, cout), F32),
        ],
    )(a, b, Wr, Ws, br2d)


def _pool_body(a_ref, b_ref, batch_ref, sums_ref, cnts_ref):
    i = pl.program_id(0)

    @pl.when(i == 0)
    def _():
        sums_ref[...] = jnp.zeros_like(sums_ref)
        cnts_ref[...] = jnp.zeros_like(cnts_ref)

    h = jnp.maximum(a_ref[...] + b_ref[...], 0.0)
    bids = batch_ref[...]
    valid = (bids < NGRAPH).astype(F32)
    onehot = jnp.where(
        lax.broadcasted_iota(jnp.int32, (NGRAPH, BN), 0) == bids, 1.0, 0.0
    ).astype(F32)
    h = jnp.where(bids.reshape(BN, 1) < NGRAPH, h, 0.0)
    sums_ref[...] += jnp.dot(onehot, h, preferred_element_type=F32)
    cnts_ref[...] += jnp.dot(
        onehot, jnp.broadcast_to(valid.reshape(BN, 1), (BN, 128)),
        preferred_element_type=F32)


def _pool(a, b, batch2d):
    return pl.pallas_call(
        _pool_body,
        grid=(NBLK,),
        in_specs=[
            pl.BlockSpec((BN, 128), lambda i: (i, 0)),
            pl.BlockSpec((BN, 128), lambda i: (i, 0)),
            pl.BlockSpec((1, BN), lambda i: (0, i)),
        ],
        out_specs=[
            pl.BlockSpec((NGRAPH, 128), lambda i: (0, 0)),
            pl.BlockSpec((NGRAPH, 128), lambda i: (0, 0)),
        ],
        out_shape=[
            jax.ShapeDtypeStruct((NGRAPH, 128), F32),
            jax.ShapeDtypeStruct((NGRAPH, 128), F32),
        ],
    )(a, b, batch2d)


def _head_body(sums_ref, cnts_ref, w0_ref, b0_ref, w1_ref, b1_ref, w2_ref,
               b2_ref, out_ref):
    g = sums_ref[...] / jnp.maximum(cnts_ref[...], 1.0)
    g = jnp.maximum(jnp.dot(g, w0_ref[...], preferred_element_type=F32)
                    + b0_ref[...], 0.0)
    g = jnp.maximum(jnp.dot(g, w1_ref[...], preferred_element_type=F32)
                    + b1_ref[...], 0.0)
    out_ref[...] = jnp.dot(g, w2_ref[...], preferred_element_type=F32) + b2_ref[...]


def _head(sums, cnts, MW0, Mb0, MW1p, Mb1p, MW2p, Mb2p):
    return pl.pallas_call(
        _head_body,
        out_shape=jax.ShapeDtypeStruct((NGRAPH, 128), F32),
    )(sums, cnts, MW0, Mb0, MW1p, Mb1p, MW2p, Mb2p)


def kernel(x, edge_index, edge_attr, batch, Wr0, br0, Ws0, Wr1, br1, Ws1,
           Wr2, br2, Ws2, Wr3, br3, Ws3, Wr4, br4, Ws4, MW0, Mb0, MW1, Mb1,
           MW2, Mb2):
    src, dst = edge_index[0], edge_index[1]
    x_pad = jnp.pad(x, ((0, NP - NNODE), (0, 0)))
    batch_pad = jnp.pad(batch, (0, NP - NNODE), constant_values=NGRAPH)
    batch2d = batch_pad.reshape(1, NP)

    keys, wts, cnts = _bucket_kernel(src, dst, edge_attr)

    def _aggregate(t, width):
        npass = 2 if width == 512 else 1
        tin = t.reshape(NP * npass, width // npass)
        out = _make_agg(width)(tin, keys, wts, cnts)
        return out

    h = x_pad
    for Wr, br, Ws in ((Wr0, br0, Ws0), (Wr1, br1, Ws1)):
        agg = _aggregate(h, Wr.shape[0]).reshape(NP, Wr.shape[0])
        h = _layer_full(agg, h, Wr, Ws, br.reshape(1, -1))
    agg2 = _aggregate(h, 512)
    h = _layer_full2(agg2[0].reshape(NP, 256), agg2[1].reshape(NP, 256), h, Wr2, Ws2, br2.reshape(1, -1))

    zeros3 = jnp.zeros_like(h)
    t3, r3 = _pre(h, zeros3, Wr3, Ws3, br3.reshape(1, -1))
    aggw3 = _aggregate(t3, Wr3.shape[1]).reshape(NP, Wr3.shape[1])
    t4, r4 = _pre(aggw3, r3, Wr4, Ws4, br4.reshape(1, -1))
    aggw4 = _aggregate(t4, Wr4.shape[1]).reshape(NP, Wr4.shape[1])

    sums, cnts = _pool(aggw4, r4, batch2d)
    MW1p = jnp.pad(MW1, ((0, 0), (0, 64)))
    Mb1p = jnp.pad(Mb1, (0, 64)).reshape(1, 128)
    MW2p = jnp.pad(MW2, ((0, 64), (0, 127)))
    Mb2p = jnp.pad(Mb2, (0, 127)).reshape(1, 128)
    out = _head(sums, cnts, MW0, Mb0.reshape(1, 128), MW1p, Mb1p, MW2p, Mb2p)
    return out[:, :1]
